# baseline (speedup 1.0000x reference)
import jax
import jax.numpy as jnp
from jax import lax
from jax.experimental import pallas as pl
from jax.experimental.pallas import tpu as pltpu

LN_EPS = 1e-5
GEPS = 1e-4
W = 8


def _scan_kernel(T, x_ref, wv_ref, bv_ref, wg_ref, bg_ref, a_ref,
                 out_ref, state_ref, val_ref, gw_ref):
    d = pl.program_id(0)
    c = pl.program_id(1)
    B = x_ref.shape[0]
    D = x_ref.shape[2]
    NS = a_ref.shape[1]
    NW = T // W

    @pl.when(c == 0)
    def _():
        state_ref[...] = jnp.zeros_like(state_ref)

    xf = x_ref[...].reshape(B * T, D)
    val = jnp.dot(xf, wv_ref[0], preferred_element_type=jnp.float32) + bv_ref[0]
    gw = jax.nn.sigmoid(
        jnp.dot(val, wg_ref[0], preferred_element_type=jnp.float32) + bg_ref[0]
    )
    val_ref[...] = val.reshape(B, T, D)
    gw_ref[...] = gw.reshape(B, T, NS)

    A3 = a_ref[...]
    A3b = A3.astype(jnp.bfloat16)
    fwd = d == 0
    tif = lax.broadcasted_iota(jnp.int32, (1, W, W * NS), 1)
    kif = lax.broadcasted_iota(jnp.int32, (1, W, W * NS), 2) // NS
    maskflat = jnp.where(fwd, (tif >= kif).astype(jnp.float32),
                         (kif >= tif).astype(jnp.float32))
    blockmask = (lax.broadcasted_iota(jnp.int32, (1, W, W * NS), 2) // NS
                 == lax.broadcasted_iota(jnp.int32, (1, W, W * NS), 1)
                 ).astype(jnp.float32)

    UW = 8

    def one_window(wi, states):
        o0 = wi * W
        gwin = gw_ref[:, pl.ds(o0, W), :]
        vwin = val_ref[:, pl.ds(o0, W), :]
        gc = jnp.maximum(gwin, GEPS)
        CP = gc
        for s in (1, 2, 4):
            prev = jnp.concatenate(
                [jnp.ones((B, s, NS), jnp.float32), CP[:, :W - s, :]], axis=1)
            CP = CP * prev
        SP = CP * pl.reciprocal(gc)
        CPlast = CP[:, W - 1:W, :]

        Tpart = gwin * jnp.where(fwd, CP, pl.reciprocal(SP))
        Kpart = (1.0 - gwin) * jnp.where(fwd, pl.reciprocal(CP), SP)
        q = jnp.where(fwd, Tpart, Tpart * CPlast)
        wk = jnp.where(fwd, Kpart * CPlast, Kpart)

        Trep = pltpu.repeat(Tpart, W, axis=2)
        Krep = pltpu.repeat(Kpart, W, axis=2)
        Kflat = jnp.sum(Krep * blockmask, axis=1, keepdims=True)
        Mflat = (Trep * Kflat * maskflat).astype(jnp.bfloat16)

        decayT = jnp.swapaxes(CPlast, 1, 2)
        vwinb = vwin.astype(jnp.bfloat16)
        new_states = []
        for b in range(B):
            av_b = (vwinb[b][:, None, :] * A3b[0][None, :, :]).reshape(W * NS, D)
            o_b = jnp.dot(Mflat[b], av_b,
                          preferred_element_type=jnp.float32)
            o_b = o_b + jnp.dot(q[b], states[b],
                                preferred_element_type=jnp.float32)
            out_ref[0, b, pl.ds(o0, W), :] = o_b
            h_b = jax.lax.dot_general(
                wk[b], vwin[b], (((0,), (0,)), ((), ())),
                preferred_element_type=jnp.float32) * A3[0]
            new_states.append(states[b] * decayT[b] + h_b)
        return tuple(new_states)

    def body(u, states):
        for j in range(UW):
            w = u * UW + j
            wi = lax.select(fwd, w, NW - 1 - w)
            states = one_window(wi, states)
        return states

    st0 = state_ref[...]
    states = lax.fori_loop(0, NW // UW, body,
                           tuple(st0[b] for b in range(B)))
    state_ref[...] = jnp.stack(states, axis=0)


def _combine_kernel(f_ref, b_ref, w1_ref, w2_ref, bo_ref, lg_ref, lb_ref, o_ref):
    h = jnp.dot(f_ref[...], w1_ref[...], preferred_element_type=jnp.float32)
    h = h + jnp.dot(b_ref[...], w2_ref[...], preferred_element_type=jnp.float32)
    h = h + bo_ref[...]
    mu = jnp.mean(h, axis=1, keepdims=True)
    xc = h - mu
    var = jnp.mean(xc * xc, axis=1, keepdims=True)
    o_ref[...] = xc * lax.rsqrt(var + LN_EPS) * lg_ref[...] + lb_ref[...]


def kernel(x, W_fproj, b_fproj, A_f, W_fgate, b_fgate,
           W_bproj, b_bproj, A_b, W_bgate, b_bgate,
           W_out, b_out, ln_g, ln_b):
    B, S, D = x.shape
    NS = A_f.shape[0]
    T = 256 if S % 256 == 0 else S
    C = S // T

    Wv = jnp.stack([W_fproj[:, D:], W_bproj[:, D:]])
    bv = jnp.stack([b_fproj[D:], b_bproj[D:]]).reshape(2, 1, D)
    Wg = jnp.stack([W_fgate, W_bgate])
    bg = jnp.stack([b_fgate, b_bgate]).reshape(2, 1, NS)
    A2 = jnp.stack([A_f, A_b])

    def x_map(d, c):
        return (0, jnp.where(d == 0, c, C - 1 - c), 0)

    def w_map(d, c):
        return (d, 0, 0)

    fb = pl.pallas_call(
        lambda *refs: _scan_kernel(T, *refs),
        grid=(2, C),
        in_specs=[
            pl.BlockSpec((B, T, D), x_map),
            pl.BlockSpec((1, D, D), w_map),
            pl.BlockSpec((1, 1, D), w_map),
            pl.BlockSpec((1, D, NS), w_map),
            pl.BlockSpec((1, 1, NS), w_map),
            pl.BlockSpec((1, NS, D), w_map),
        ],
        out_specs=pl.BlockSpec(
            (1, B, T, D), lambda d, c: (d, 0, jnp.where(d == 0, c, C - 1 - c), 0)
        ),
        out_shape=jax.ShapeDtypeStruct((2, B, S, D), jnp.float32),
        scratch_shapes=[
            pltpu.VMEM((B, NS, D), jnp.float32),
            pltpu.VMEM((B, T, D), jnp.float32),
            pltpu.VMEM((B, T, NS), jnp.float32),
        ],
        compiler_params=pltpu.CompilerParams(
            dimension_semantics=("parallel", "arbitrary"),
        ),
    )(x, Wv, bv, Wg, bg, A2)

    fwd = fb[0].reshape(B * S, D)
    bwd = fb[1].reshape(B * S, D)

    T2 = 512 if (B * S) % 512 == 0 else B * S
    M = (B * S) // T2
    out = pl.pallas_call(
        _combine_kernel,
        grid=(M,),
        in_specs=[
            pl.BlockSpec((T2, D), lambda i: (i, 0)),
            pl.BlockSpec((T2, D), lambda i: (i, 0)),
            pl.BlockSpec((D, D), lambda i: (0, 0)),
            pl.BlockSpec((D, D), lambda i: (0, 0)),
            pl.BlockSpec((1, D), lambda i: (0, 0)),
            pl.BlockSpec((1, D), lambda i: (0, 0)),
            pl.BlockSpec((1, D), lambda i: (0, 0)),
        ],
        out_specs=pl.BlockSpec((T2, D), lambda i: (i, 0)),
        out_shape=jax.ShapeDtypeStruct((B * S, D), jnp.float32),
        compiler_params=pltpu.CompilerParams(
            dimension_semantics=("parallel",),
        ),
    )(fwd, bwd, W_out[:D], W_out[D:], b_out.reshape(1, D),
      ln_g.reshape(1, D), ln_b.reshape(1, D))

    return out.reshape(B, S, D)


# speedup vs baseline: 1.1587x; 1.1587x over previous
import jax
import jax.numpy as jnp
from jax import lax
from jax.experimental import pallas as pl
from jax.experimental.pallas import tpu as pltpu

LN_EPS = 1e-5
GEPS = 1e-4
W = 8


def _scan_kernel(T, x_ref, wv_ref, bv_ref, wg_ref, bg_ref, a_ref,
                 out_ref, state_ref, val_ref, gw_ref):
    d = pl.program_id(0)
    c = pl.program_id(1)
    B = x_ref.shape[0]
    D = x_ref.shape[2]
    NS = a_ref.shape[1]
    NW = T // W

    @pl.when(c == 0)
    def _():
        state_ref[...] = jnp.zeros_like(state_ref)

    xf = x_ref[...].reshape(B * T, D)
    val = jnp.dot(xf, wv_ref[0], preferred_element_type=jnp.float32) + bv_ref[0]
    gw = jax.nn.sigmoid(
        jnp.dot(val, wg_ref[0], preferred_element_type=jnp.float32) + bg_ref[0]
    )
    val_ref[...] = val.reshape(B, T, D)
    gw_ref[...] = gw.reshape(B, T, NS)

    A3 = a_ref[...]
    fwd = d == 0
    tif = lax.broadcasted_iota(jnp.int32, (1, W, W * NS), 1)
    kif = lax.broadcasted_iota(jnp.int32, (1, W, W * NS), 2) // NS
    maskflat = jnp.where(fwd, (tif >= kif).astype(jnp.float32),
                         (kif >= tif).astype(jnp.float32))
    blockmask = (lax.broadcasted_iota(jnp.int32, (1, W, W * NS), 2) // NS
                 == lax.broadcasted_iota(jnp.int32, (1, W, W * NS), 1)
                 ).astype(jnp.float32)

    UW = 8

    def one_window(wi, states):
        o0 = wi * W
        gwin = gw_ref[:, pl.ds(o0, W), :]
        vwin = val_ref[:, pl.ds(o0, W), :]
        gc = jnp.maximum(gwin, GEPS)
        CP = gc
        for s in (1, 2, 4):
            prev = jnp.concatenate(
                [jnp.ones((B, s, NS), jnp.float32), CP[:, :W - s, :]], axis=1)
            CP = CP * prev
        SP = CP * pl.reciprocal(gc)
        CPlast = CP[:, W - 1:W, :]

        Tpart = gwin * jnp.where(fwd, CP, pl.reciprocal(SP))
        Kpart = (1.0 - gwin) * jnp.where(fwd, pl.reciprocal(CP), SP)
        q = jnp.where(fwd, Tpart, Tpart * CPlast)
        wk = jnp.where(fwd, Kpart * CPlast, Kpart)

        Trep = pltpu.repeat(Tpart, W, axis=2)
        Krep = pltpu.repeat(Kpart, W, axis=2)
        Kflat = jnp.sum(Krep * blockmask, axis=1, keepdims=True)
        Mflat = Trep * Kflat * maskflat

        decayT = jnp.swapaxes(CPlast, 1, 2)
        new_states = []
        for b in range(B):
            av_b = (vwin[b][:, None, :] * A3[0][None, :, :]).reshape(W * NS, D)
            o_b = jnp.dot(Mflat[b], av_b,
                          preferred_element_type=jnp.float32)
            o_b = o_b + jnp.dot(q[b], states[b],
                                preferred_element_type=jnp.float32)
            out_ref[0, b, pl.ds(o0, W), :] = o_b
            h_b = jax.lax.dot_general(
                wk[b], vwin[b], (((0,), (0,)), ((), ())),
                preferred_element_type=jnp.float32) * A3[0]
            new_states.append(states[b] * decayT[b] + h_b)
        return tuple(new_states)

    st0 = state_ref[...]
    states = tuple(st0[b] for b in range(B))
    for w in range(NW):
        wi = lax.select(fwd, w, NW - 1 - w)
        states = one_window(wi, states)
    state_ref[...] = jnp.stack(states, axis=0)


def _combine_kernel(f_ref, b_ref, w1_ref, w2_ref, bo_ref, lg_ref, lb_ref, o_ref):
    h = jnp.dot(f_ref[...], w1_ref[...], preferred_element_type=jnp.float32)
    h = h + jnp.dot(b_ref[...], w2_ref[...], preferred_element_type=jnp.float32)
    h = h + bo_ref[...]
    mu = jnp.mean(h, axis=1, keepdims=True)
    xc = h - mu
    var = jnp.mean(xc * xc, axis=1, keepdims=True)
    o_ref[...] = xc * lax.rsqrt(var + LN_EPS) * lg_ref[...] + lb_ref[...]


def kernel(x, W_fproj, b_fproj, A_f, W_fgate, b_fgate,
           W_bproj, b_bproj, A_b, W_bgate, b_bgate,
           W_out, b_out, ln_g, ln_b):
    B, S, D = x.shape
    NS = A_f.shape[0]
    T = 256 if S % 256 == 0 else S
    C = S // T

    Wv = jnp.stack([W_fproj[:, D:], W_bproj[:, D:]])
    bv = jnp.stack([b_fproj[D:], b_bproj[D:]]).reshape(2, 1, D)
    Wg = jnp.stack([W_fgate, W_bgate])
    bg = jnp.stack([b_fgate, b_bgate]).reshape(2, 1, NS)
    A2 = jnp.stack([A_f, A_b])

    def x_map(d, c):
        return (0, jnp.where(d == 0, c, C - 1 - c), 0)

    def w_map(d, c):
        return (d, 0, 0)

    fb = pl.pallas_call(
        lambda *refs: _scan_kernel(T, *refs),
        grid=(2, C),
        in_specs=[
            pl.BlockSpec((B, T, D), x_map),
            pl.BlockSpec((1, D, D), w_map),
            pl.BlockSpec((1, 1, D), w_map),
            pl.BlockSpec((1, D, NS), w_map),
            pl.BlockSpec((1, 1, NS), w_map),
            pl.BlockSpec((1, NS, D), w_map),
        ],
        out_specs=pl.BlockSpec(
            (1, B, T, D), lambda d, c: (d, 0, jnp.where(d == 0, c, C - 1 - c), 0)
        ),
        out_shape=jax.ShapeDtypeStruct((2, B, S, D), jnp.float32),
        scratch_shapes=[
            pltpu.VMEM((B, NS, D), jnp.float32),
            pltpu.VMEM((B, T, D), jnp.float32),
            pltpu.VMEM((B, T, NS), jnp.float32),
        ],
        compiler_params=pltpu.CompilerParams(
            dimension_semantics=("parallel", "arbitrary"),
        ),
    )(x, Wv, bv, Wg, bg, A2)

    fwd = fb[0].reshape(B * S, D)
    bwd = fb[1].reshape(B * S, D)

    T2 = 512 if (B * S) % 512 == 0 else B * S
    M = (B * S) // T2
    out = pl.pallas_call(
        _combine_kernel,
        grid=(M,),
        in_specs=[
            pl.BlockSpec((T2, D), lambda i: (i, 0)),
            pl.BlockSpec((T2, D), lambda i: (i, 0)),
            pl.BlockSpec((D, D), lambda i: (0, 0)),
            pl.BlockSpec((D, D), lambda i: (0, 0)),
            pl.BlockSpec((1, D), lambda i: (0, 0)),
            pl.BlockSpec((1, D), lambda i: (0, 0)),
            pl.BlockSpec((1, D), lambda i: (0, 0)),
        ],
        out_specs=pl.BlockSpec((T2, D), lambda i: (i, 0)),
        out_shape=jax.ShapeDtypeStruct((B * S, D), jnp.float32),
        compiler_params=pltpu.CompilerParams(
            dimension_semantics=("parallel",),
        ),
    )(fwd, bwd, W_out[:D], W_out[D:], b_out.reshape(1, D),
      ln_g.reshape(1, D), ln_b.reshape(1, D))

    return out.reshape(B, S, D)


# speedup vs baseline: 1.2106x; 1.0449x over previous
import jax
import jax.numpy as jnp
from jax import lax
from jax.experimental import pallas as pl
from jax.experimental.pallas import tpu as pltpu

LN_EPS = 1e-5
GEPS = 1e-4
W = 8


def _scan_kernel(T, x_ref, wv_ref, bv_ref, wg_ref, bg_ref, a_ref,
                 out_ref, state_ref, val_ref, gw_ref):
    d = pl.program_id(0)
    c = pl.program_id(1)
    B = x_ref.shape[0]
    D = x_ref.shape[2]
    NS = a_ref.shape[1]
    NW = T // W

    @pl.when(c == 0)
    def _():
        state_ref[...] = jnp.zeros_like(state_ref)

    xf = x_ref[...].reshape(B * T, D)
    val = jnp.dot(xf, wv_ref[0], preferred_element_type=jnp.float32) + bv_ref[0]
    gw = jax.nn.sigmoid(
        jnp.dot(val, wg_ref[0], preferred_element_type=jnp.float32) + bg_ref[0]
    )
    val_ref[...] = val.reshape(B, T, D)
    gw_ref[...] = gw.reshape(B, T, NS)

    A3 = a_ref[...]
    fwd = d == 0
    tif = lax.broadcasted_iota(jnp.int32, (1, W, W * NS), 1)
    kif = lax.broadcasted_iota(jnp.int32, (1, W, W * NS), 2) // NS
    maskflat = jnp.where(fwd, (tif >= kif).astype(jnp.float32),
                         (kif >= tif).astype(jnp.float32))
    blockmask = (lax.broadcasted_iota(jnp.int32, (1, W, W * NS), 2) // NS
                 == lax.broadcasted_iota(jnp.int32, (1, W, W * NS), 1)
                 ).astype(jnp.float32)

    UW = 8

    def one_window(wi, states):
        o0 = wi * W
        gwin = gw_ref[:, pl.ds(o0, W), :]
        vwin = val_ref[:, pl.ds(o0, W), :]
        gc = jnp.maximum(gwin, GEPS)
        CP = gc
        for s in (1, 2, 4):
            prev = jnp.concatenate(
                [jnp.ones((B, s, NS), jnp.float32), CP[:, :W - s, :]], axis=1)
            CP = CP * prev
        SP = CP * pl.reciprocal(gc)
        CPlast = CP[:, W - 1:W, :]

        Tpart = gwin * jnp.where(fwd, CP, pl.reciprocal(SP))
        Kpart = (1.0 - gwin) * jnp.where(fwd, pl.reciprocal(CP), SP)
        q = jnp.where(fwd, Tpart, Tpart * CPlast)
        wk = jnp.where(fwd, Kpart * CPlast, Kpart)

        Trep = pltpu.repeat(Tpart, W, axis=2)
        Krep = pltpu.repeat(Kpart, W, axis=2)
        Kflat = jnp.sum(Krep * blockmask, axis=1, keepdims=True)
        Mflat = Trep * Kflat * maskflat

        decayT = jnp.swapaxes(CPlast, 1, 2)
        new_states = []
        for b in range(B):
            av_b = (vwin[b][:, None, :] * A3[0][None, :, :]).reshape(W * NS, D)
            o_b = jnp.dot(Mflat[b], av_b,
                          preferred_element_type=jnp.float32)
            o_b = o_b + jnp.dot(q[b], states[b],
                                preferred_element_type=jnp.float32)
            out_ref[0, b, pl.ds(o0, W), :] = o_b
            h_b = jax.lax.dot_general(
                wk[b], vwin[b], (((0,), (0,)), ((), ())),
                preferred_element_type=jnp.float32) * A3[0]
            new_states.append(states[b] * decayT[b] + h_b)
        return tuple(new_states)

    st0 = state_ref[...]
    states = tuple(st0[b] for b in range(B))
    for w in range(NW):
        wi = lax.select(fwd, w, NW - 1 - w)
        states = one_window(wi, states)
    state_ref[...] = jnp.stack(states, axis=0)


def _combine_kernel(f_ref, b_ref, w1_ref, w2_ref, bo_ref, lg_ref, lb_ref, o_ref):
    h = jnp.dot(f_ref[...], w1_ref[...], preferred_element_type=jnp.float32)
    h = h + jnp.dot(b_ref[...], w2_ref[...], preferred_element_type=jnp.float32)
    h = h + bo_ref[...]
    mu = jnp.mean(h, axis=1, keepdims=True)
    xc = h - mu
    var = jnp.mean(xc * xc, axis=1, keepdims=True)
    o_ref[...] = xc * lax.rsqrt(var + LN_EPS) * lg_ref[...] + lb_ref[...]


def kernel(x, W_fproj, b_fproj, A_f, W_fgate, b_fgate,
           W_bproj, b_bproj, A_b, W_bgate, b_bgate,
           W_out, b_out, ln_g, ln_b):
    B, S, D = x.shape
    NS = A_f.shape[0]
    T = 512 if S % 512 == 0 else S
    C = S // T

    Wv = jnp.stack([W_fproj[:, D:], W_bproj[:, D:]])
    bv = jnp.stack([b_fproj[D:], b_bproj[D:]]).reshape(2, 1, D)
    Wg = jnp.stack([W_fgate, W_bgate])
    bg = jnp.stack([b_fgate, b_bgate]).reshape(2, 1, NS)
    A2 = jnp.stack([A_f, A_b])

    def x_map(d, c):
        return (0, jnp.where(d == 0, c, C - 1 - c), 0)

    def w_map(d, c):
        return (d, 0, 0)

    fb = pl.pallas_call(
        lambda *refs: _scan_kernel(T, *refs),
        grid=(2, C),
        in_specs=[
            pl.BlockSpec((B, T, D), x_map),
            pl.BlockSpec((1, D, D), w_map),
            pl.BlockSpec((1, 1, D), w_map),
            pl.BlockSpec((1, D, NS), w_map),
            pl.BlockSpec((1, 1, NS), w_map),
            pl.BlockSpec((1, NS, D), w_map),
        ],
        out_specs=pl.BlockSpec(
            (1, B, T, D), lambda d, c: (d, 0, jnp.where(d == 0, c, C - 1 - c), 0)
        ),
        out_shape=jax.ShapeDtypeStruct((2, B, S, D), jnp.float32),
        scratch_shapes=[
            pltpu.VMEM((B, NS, D), jnp.float32),
            pltpu.VMEM((B, T, D), jnp.float32),
            pltpu.VMEM((B, T, NS), jnp.float32),
        ],
        compiler_params=pltpu.CompilerParams(
            dimension_semantics=("parallel", "arbitrary"),
        ),
    )(x, Wv, bv, Wg, bg, A2)

    fwd = fb[0].reshape(B * S, D)
    bwd = fb[1].reshape(B * S, D)

    T2 = 512 if (B * S) % 512 == 0 else B * S
    M = (B * S) // T2
    out = pl.pallas_call(
        _combine_kernel,
        grid=(M,),
        in_specs=[
            pl.BlockSpec((T2, D), lambda i: (i, 0)),
            pl.BlockSpec((T2, D), lambda i: (i, 0)),
            pl.BlockSpec((D, D), lambda i: (0, 0)),
            pl.BlockSpec((D, D), lambda i: (0, 0)),
            pl.BlockSpec((1, D), lambda i: (0, 0)),
            pl.BlockSpec((1, D), lambda i: (0, 0)),
            pl.BlockSpec((1, D), lambda i: (0, 0)),
        ],
        out_specs=pl.BlockSpec((T2, D), lambda i: (i, 0)),
        out_shape=jax.ShapeDtypeStruct((B * S, D), jnp.float32),
        compiler_params=pltpu.CompilerParams(
            dimension_semantics=("parallel",),
        ),
    )(fwd, bwd, W_out[:D], W_out[D:], b_out.reshape(1, D),
      ln_g.reshape(1, D), ln_b.reshape(1, D))

    return out.reshape(B, S, D)
